# baseline (speedup 1.0000x reference)
import functools

import jax
import jax.numpy as jnp
from jax import lax
from jax.experimental import pallas as pl
from jax.experimental.pallas import tpu as pltpu
from jax.experimental.pallas import tpu_sc as plsc

B, IMG, TXT, D = 8, 4096, 512, 2048
BLK = 1024
NCHUNK = 4
CW = D // NCHUNK
RCH = 64
NV = CW // 16


def _sc_mean_kernel(enc_hbm, mask_hbm, out_hbm, mask_v, rows_v, acc_v):
    nc = 2
    wid = lax.axis_index("s") * nc + lax.axis_index("c")
    b = wid // NCHUNK
    c = wid % NCHUNK
    col0 = c * CW

    pltpu.sync_copy(mask_hbm.at[b], mask_v)

    def cnt_body(v, c):
        kvec = jnp.where(mask_v[pl.ds(v * 16, 16)] > -9000.0, 1.0, 0.0)
        for r in range(16):
            c = c + kvec[r]
        return c

    cnt = lax.fori_loop(0, TXT // 16, cnt_body, jnp.float32(0.0))

    for v in range(NV):
        acc_v[pl.ds(v * 16, 16)] = jnp.zeros((16,), jnp.float32)

    def chunk_body(g, _):
        pltpu.sync_copy(
            enc_hbm.at[b, pl.ds(g * RCH, RCH), pl.ds(col0, CW)], rows_v)

        def group_body(rg, _):
            mvec = mask_v[pl.ds(g * RCH + rg * 16, 16)]
            kvec = jnp.where(mvec > -9000.0, 1.0, 0.0)
            row0 = rg * 16
            for v in range(NV):
                a = acc_v[pl.ds(v * 16, 16)]
                for r16 in range(16):
                    a = a + rows_v[row0 + r16, pl.ds(v * 16, 16)] * kvec[r16]
                acc_v[pl.ds(v * 16, 16)] = a
            return 0

        lax.fori_loop(0, RCH // 16, group_body, 0)
        return 0

    lax.fori_loop(0, TXT // RCH, chunk_body, 0)

    cnt_vec = jnp.full((16,), cnt, jnp.float32)
    inv = jnp.ones((16,), jnp.float32) / jnp.maximum(cnt_vec, 1.0)
    for v in range(NV):
        acc_v[pl.ds(v * 16, 16)] = acc_v[pl.ds(v * 16, 16)] * inv

    pltpu.sync_copy(acc_v, out_hbm.at[b, pl.ds(col0, CW)])


def _tc_body(mean_ref, ts_ref, hs_ref, gamma_ref, tproj_ref, out_ref):
    t_emb = ts_ref[0, 0, 0] * tproj_ref[0, :] * 0.001
    add = (mean_ref[0, 0, :] + t_emb)[None, :]
    out_ref[0] = hs_ref[0] * gamma_ref[0, :][None, :] + add


@jax.jit
def _run(hidden_states, encoder_hidden_states, mask2d, gamma, t_proj, ts_f):
    mean = pl.kernel(
        _sc_mean_kernel,
        out_type=jax.ShapeDtypeStruct((B, D), jnp.float32),
        mesh=plsc.VectorSubcoreMesh(core_axis_name="c", subcore_axis_name="s"),
        scratch_types=[
            pltpu.VMEM((TXT,), jnp.float32),
            pltpu.VMEM((RCH, CW), jnp.float32),
            pltpu.VMEM((CW,), jnp.float32),
        ],
    )(encoder_hidden_states, mask2d)

    grid = (B, IMG // BLK)
    return pl.pallas_call(
        _tc_body,
        grid=grid,
        in_specs=[
            pl.BlockSpec((1, 1, D), lambda b, j: (b, 0, 0)),
            pl.BlockSpec((1, 1, 1), lambda b, j: (b, 0, 0)),
            pl.BlockSpec((1, BLK, D), lambda b, j: (b, j, 0)),
            pl.BlockSpec((1, D), lambda b, j: (0, 0)),
            pl.BlockSpec((1, D), lambda b, j: (0, 0)),
        ],
        out_specs=pl.BlockSpec((1, BLK, D), lambda b, j: (b, j, 0)),
        out_shape=jax.ShapeDtypeStruct((B, IMG, D), jnp.float32),
        compiler_params=pltpu.CompilerParams(
            dimension_semantics=("arbitrary", "arbitrary"),
        ),
    )(mean.reshape(B, 1, D), ts_f, hidden_states,
      gamma.reshape(1, D), t_proj.reshape(1, D))


def kernel(hidden_states, encoder_hidden_states, encoder_attention_mask,
           gamma, t_proj, timestep):
    ts_f = timestep.astype(jnp.float32).reshape(B, 1, 1)
    mask2d = encoder_attention_mask.reshape(B, TXT)
    return _run(hidden_states, encoder_hidden_states, mask2d,
                gamma, t_proj, ts_f)


# speedup vs baseline: 1.3080x; 1.3080x over previous
import jax
import jax.numpy as jnp
from jax.experimental import pallas as pl
from jax.experimental.pallas import tpu as pltpu

B, IMG, TXT, D = 8, 4096, 512, 2048
BLK = 1024


def _fused_body(mask_ref, enc_ref, ts_ref, hs_ref, gamma_ref, tproj_ref,
                out_ref, mean_ref):
    j = pl.program_id(1)

    @pl.when(j == 0)
    def _compute_mean():
        m = mask_ref[0, 0, :] > -9000.0
        cnt = jnp.sum(m.astype(jnp.float32))
        keepf = m.astype(jnp.float32)[:, None]
        s = jnp.sum(enc_ref[0] * keepf, axis=0)
        t_emb = ts_ref[0, 0, 0] * tproj_ref[0, :] * 0.001
        mean_ref[0, :] = s / jnp.maximum(cnt, 1.0) + t_emb

    add = mean_ref[0, :][None, :]
    out_ref[0] = hs_ref[0] * gamma_ref[0, :][None, :] + add


@jax.jit
def _run(hidden_states, encoder_hidden_states, encoder_attention_mask,
         gamma, t_proj, ts_f):
    grid = (B, IMG // BLK)
    return pl.pallas_call(
        _fused_body,
        grid=grid,
        in_specs=[
            pl.BlockSpec((1, 1, TXT), lambda b, j: (b, 0, 0)),
            pl.BlockSpec((1, TXT, D), lambda b, j: (b, 0, 0)),
            pl.BlockSpec((1, 1, 1), lambda b, j: (b, 0, 0)),
            pl.BlockSpec((1, BLK, D), lambda b, j: (b, j, 0)),
            pl.BlockSpec((1, D), lambda b, j: (0, 0)),
            pl.BlockSpec((1, D), lambda b, j: (0, 0)),
        ],
        out_specs=pl.BlockSpec((1, BLK, D), lambda b, j: (b, j, 0)),
        out_shape=jax.ShapeDtypeStruct((B, IMG, D), jnp.float32),
        scratch_shapes=[pltpu.VMEM((1, D), jnp.float32)],
        compiler_params=pltpu.CompilerParams(
            dimension_semantics=("parallel", "arbitrary"),
        ),
    )(encoder_attention_mask, encoder_hidden_states, ts_f,
      hidden_states, gamma.reshape(1, D), t_proj.reshape(1, D))


def kernel(hidden_states, encoder_hidden_states, encoder_attention_mask,
           gamma, t_proj, timestep):
    ts_f = timestep.astype(jnp.float32).reshape(B, 1, 1)
    return _run(hidden_states, encoder_hidden_states, encoder_attention_mask,
                gamma, t_proj, ts_f)
